# baseline (device time: 175784 ns/iter reference)
import jax
import jax.numpy as jnp
from jax import lax
from jax.experimental import pallas as pl
from jax.experimental.pallas import tpu as pltpu

N_DEV = 8
N_HOPS = N_DEV - 1
NSLOT = 3
S_LOC = 1024
H = 8
HH = H // 2
D = 128
D_MODEL = H * D
ROWS = 2 * H
SCALE = 0.08838834764831843
Q_SCALE = 3.5 / 127.0


def _attn_body(q_ref, kv_ref, wo_ref, out_ref,
               kvbuf, ctx_ref, l_scr, acc_scr,
               cw_send, cw_recv, ccw_send, ccw_recv, cw_cred, ccw_cred):
    my = lax.axis_index("i")
    right = lax.rem(my + 1, N_DEV)
    left = lax.rem(my + N_DEV - 1, N_DEV)

    barrier = pltpu.get_barrier_semaphore()
    for nbr in (left, right):
        pl.semaphore_signal(barrier, inc=1, device_id=(nbr,),
                            device_id_type=pl.DeviceIdType.MESH)
    pl.semaphore_wait(barrier, 2)

    def make_rdma(h, direction):
        s = (h % NSLOT) * ROWS
        ps = ((h - 1) % NSLOT) * ROWS
        if direction == 0:
            src = (kv_ref.at[pl.ds(0, H)] if h == 0
                   else kvbuf.at[pl.ds(ps, H)])
            return pltpu.make_async_remote_copy(
                src_ref=src, dst_ref=kvbuf.at[pl.ds(s, H)],
                send_sem=cw_send.at[h % NSLOT],
                recv_sem=cw_recv.at[h % NSLOT],
                device_id=(right,), device_id_type=pl.DeviceIdType.MESH)
        else:
            src = (kv_ref.at[pl.ds(H, H)] if h == 0
                   else kvbuf.at[pl.ds(ps + H, H)])
            return pltpu.make_async_remote_copy(
                src_ref=src, dst_ref=kvbuf.at[pl.ds(s + H, H)],
                send_sem=ccw_send.at[h % NSLOT],
                recv_sem=ccw_recv.at[h % NSLOT],
                device_id=(left,), device_id_type=pl.DeviceIdType.MESH)

    cw0 = make_rdma(0, 0)
    ccw0 = make_rdma(0, 1)
    cw0.start()
    ccw0.start()

    l_scr[...] = jnp.zeros((H, S_LOC, 1), jnp.float32)
    acc_scr[...] = jnp.zeros((H, S_LOC, D), jnp.float32)

    def process(dir_ref, base):
        def body(g, _):
            row_k = base + g + jnp.where(g < HH, 0, HH)
            row_v = row_k + HH
            q_h = q_ref[g]
            k_c = dir_ref[row_k].astype(jnp.bfloat16)
            v_c = dir_ref[row_v].astype(jnp.bfloat16)
            s = lax.dot_general(q_h, k_c, (((1,), (1,)), ((), ())),
                                preferred_element_type=jnp.float32)
            p = jnp.exp(s)
            l_scr[g] += jnp.sum(p, axis=-1, keepdims=True)
            acc_scr[g] += lax.dot_general(
                p.astype(jnp.bfloat16), v_c, (((1,), (0,)), ((), ())),
                preferred_element_type=jnp.float32)
            return 0
        lax.fori_loop(0, H, body, 0)

    process(kv_ref, 0)

    descs = [(cw0, ccw0)]
    for h in range(N_HOPS):
        cw, ccw = descs[h]
        cw.wait()
        ccw.wait()
        if 1 <= h <= N_HOPS - NSLOT:
            fs = (h - 1) % NSLOT
            pl.semaphore_signal(cw_cred.at[fs], inc=1, device_id=(left,),
                                device_id_type=pl.DeviceIdType.MESH)
            pl.semaphore_signal(ccw_cred.at[fs], inc=1, device_id=(right,),
                                device_id_type=pl.DeviceIdType.MESH)
        if h + 1 < N_HOPS:
            if h + 1 >= NSLOT:
                pl.semaphore_wait(cw_cred.at[(h + 1) % NSLOT], 1)
                pl.semaphore_wait(ccw_cred.at[(h + 1) % NSLOT], 1)
            nxt = (make_rdma(h + 1, 0), make_rdma(h + 1, 1))
            nxt[0].start()
            nxt[1].start()
            descs.append(nxt)
        process(kvbuf, (h % NSLOT) * ROWS)

    for g in range(H):
        ctx_ref[:, g * D:(g + 1) * D] = (
            acc_scr[g] / l_scr[g]).astype(jnp.bfloat16)

    out_ref[...] = jnp.dot(ctx_ref[...], wo_ref[...],
                           preferred_element_type=jnp.float32)


def _attention(qh, kv8, wo):
    return pl.pallas_call(
        _attn_body,
        out_shape=jax.ShapeDtypeStruct((S_LOC, D_MODEL), jnp.float32),
        in_specs=[pl.BlockSpec(memory_space=pltpu.VMEM)] * 3,
        out_specs=pl.BlockSpec(memory_space=pltpu.VMEM),
        scratch_shapes=[
            pltpu.VMEM((NSLOT * ROWS, S_LOC, D), jnp.int8),
            pltpu.VMEM((S_LOC, D_MODEL), jnp.bfloat16),
            pltpu.VMEM((H, S_LOC, 1), jnp.float32),
            pltpu.VMEM((H, S_LOC, D), jnp.float32),
            pltpu.SemaphoreType.DMA((NSLOT,)),
            pltpu.SemaphoreType.DMA((NSLOT,)),
            pltpu.SemaphoreType.DMA((NSLOT,)),
            pltpu.SemaphoreType.DMA((NSLOT,)),
            pltpu.SemaphoreType.REGULAR((NSLOT,)),
            pltpu.SemaphoreType.REGULAR((NSLOT,)),
        ],
        compiler_params=pltpu.CompilerParams(
            collective_id=0, vmem_limit_bytes=52 * 1024 * 1024),
    )(qh, kv8, wo)


def kernel(x, Wq, Wk, Wv, Wo):
    my = lax.axis_index("i")
    x2 = x[0].astype(jnp.bfloat16)

    w3 = jnp.concatenate(
        [Wq * (SCALE * Q_SCALE), Wk, Wv], axis=1).astype(jnp.bfloat16)
    qkv = jnp.dot(x2, w3, preferred_element_type=jnp.float32)
    q = qkv[:, :D_MODEL]
    k = qkv[:, D_MODEL:2 * D_MODEL]
    v = qkv[:, 2 * D_MODEL:]

    pos = (my * S_LOC + jnp.arange(S_LOC)).astype(jnp.float32)[:, None]
    inv = 1.0 / (10000.0 ** (jnp.arange(0, D, 2).astype(jnp.float32) / D))
    ang = pos * inv[None, :]
    cos = jnp.repeat(jnp.cos(ang), 2, axis=-1)
    sin = jnp.repeat(jnp.sin(ang), 2, axis=-1)

    def rot(t):
        t2 = t.reshape(S_LOC, H, D // 2, 2)
        t_r = jnp.stack([-t2[..., 1], t2[..., 0]], axis=-1).reshape(S_LOC, H, D)
        return t * cos[:, None, :] + t_r * sin[:, None, :]

    def to_i8(t):
        return jnp.clip(jnp.round(t / Q_SCALE), -127, 127).astype(jnp.int8)

    qh = rot(q.reshape(S_LOC, H, D)).transpose(1, 0, 2).astype(jnp.bfloat16)
    k8 = to_i8(rot(k.reshape(S_LOC, H, D)).transpose(1, 0, 2))
    v8 = to_i8(v.reshape(S_LOC, H, D).transpose(1, 0, 2))

    kv8 = jnp.concatenate([k8[:HH], v8[:HH], k8[HH:], v8[HH:]], axis=0)

    wo = (Wo * Q_SCALE).astype(jnp.bfloat16)
    out = _attention(qh, kv8, wo)
    return out.reshape(1, S_LOC, D_MODEL)


# device time: 172624 ns/iter; 1.0183x vs baseline; 1.0183x over previous
import jax
import jax.numpy as jnp
from jax import lax
from jax.experimental import pallas as pl
from jax.experimental.pallas import tpu as pltpu

N_DEV = 8
N_HOPS = N_DEV - 1
NSLOT = 3
S_LOC = 1024
H = 8
HH = H // 2
D = 128
D_MODEL = H * D
ROWS = 2 * H
SCALE = 0.08838834764831843
Q_SCALE = 3.5 / 127.0


def _attn_body(q_ref, kv_ref, wo_ref, out_ref,
               kvbuf, ctx_ref, l_scr, acc_scr,
               cw_send, cw_recv, ccw_send, ccw_recv, cw_cred, ccw_cred):
    my = lax.axis_index("i")
    right = lax.rem(my + 1, N_DEV)
    left = lax.rem(my + N_DEV - 1, N_DEV)

    barrier = pltpu.get_barrier_semaphore()
    for nbr in (left, right):
        pl.semaphore_signal(barrier, inc=1, device_id=(nbr,),
                            device_id_type=pl.DeviceIdType.MESH)
    pl.semaphore_wait(barrier, 2)

    def make_rdma(h, direction):
        s = (h % NSLOT) * ROWS
        ps = ((h - 1) % NSLOT) * ROWS
        if direction == 0:
            src = (kv_ref.at[pl.ds(0, H)] if h == 0
                   else kvbuf.at[pl.ds(ps, H)])
            return pltpu.make_async_remote_copy(
                src_ref=src, dst_ref=kvbuf.at[pl.ds(s, H)],
                send_sem=cw_send.at[h % NSLOT],
                recv_sem=cw_recv.at[h % NSLOT],
                device_id=(right,), device_id_type=pl.DeviceIdType.MESH)
        else:
            src = (kv_ref.at[pl.ds(H, H)] if h == 0
                   else kvbuf.at[pl.ds(ps + H, H)])
            return pltpu.make_async_remote_copy(
                src_ref=src, dst_ref=kvbuf.at[pl.ds(s + H, H)],
                send_sem=ccw_send.at[h % NSLOT],
                recv_sem=ccw_recv.at[h % NSLOT],
                device_id=(left,), device_id_type=pl.DeviceIdType.MESH)

    cw0 = make_rdma(0, 0)
    ccw0 = make_rdma(0, 1)
    cw0.start()
    ccw0.start()

    l_scr[...] = jnp.zeros((H, S_LOC, 1), jnp.float32)
    acc_scr[...] = jnp.zeros((H, S_LOC, D), jnp.float32)

    def process(dir_ref, base):
        def body(g, _):
            row_k = base + g + jnp.where(g < HH, 0, HH)
            row_v = row_k + HH
            q_h = q_ref[g]
            k_c = dir_ref[row_k].astype(jnp.bfloat16)
            v_c = dir_ref[row_v].astype(jnp.bfloat16)
            s = lax.dot_general(q_h, k_c, (((1,), (1,)), ((), ())),
                                preferred_element_type=jnp.float32)
            p = jnp.exp(s)
            l_scr[g] += jnp.sum(p, axis=-1, keepdims=True)
            acc_scr[g] += lax.dot_general(
                p.astype(jnp.bfloat16), v_c, (((1,), (0,)), ((), ())),
                preferred_element_type=jnp.float32)
            return 0
        lax.fori_loop(0, H, body, 0)

    process(kv_ref, 0)

    descs = [(cw0, ccw0)]
    for h in range(N_HOPS):
        cw, ccw = descs[h]
        cw.wait()
        ccw.wait()
        if 1 <= h <= N_HOPS - NSLOT:
            fs = (h - 1) % NSLOT
            pl.semaphore_signal(cw_cred.at[fs], inc=1, device_id=(left,),
                                device_id_type=pl.DeviceIdType.MESH)
            pl.semaphore_signal(ccw_cred.at[fs], inc=1, device_id=(right,),
                                device_id_type=pl.DeviceIdType.MESH)
        if h + 1 < N_HOPS:
            if h + 1 >= NSLOT:
                pl.semaphore_wait(cw_cred.at[(h + 1) % NSLOT], 1)
                pl.semaphore_wait(ccw_cred.at[(h + 1) % NSLOT], 1)
            nxt = (make_rdma(h + 1, 0), make_rdma(h + 1, 1))
            nxt[0].start()
            nxt[1].start()
            descs.append(nxt)
        process(kvbuf, (h % NSLOT) * ROWS)

    for g in range(H):
        ctx_ref[:, g * D:(g + 1) * D] = (
            acc_scr[g] / l_scr[g]).astype(jnp.bfloat16)

    out_ref[...] = jnp.dot(ctx_ref[...], wo_ref[...],
                           preferred_element_type=jnp.float32)


def _attention(qh, kv8, wo):
    return pl.pallas_call(
        _attn_body,
        out_shape=jax.ShapeDtypeStruct((S_LOC, D_MODEL), jnp.float32),
        in_specs=[pl.BlockSpec(memory_space=pltpu.VMEM)] * 3,
        out_specs=pl.BlockSpec(memory_space=pltpu.VMEM),
        scratch_shapes=[
            pltpu.VMEM((NSLOT * ROWS, S_LOC, D), jnp.int8),
            pltpu.VMEM((S_LOC, D_MODEL), jnp.bfloat16),
            pltpu.VMEM((H, S_LOC, 1), jnp.float32),
            pltpu.VMEM((H, S_LOC, D), jnp.float32),
            pltpu.SemaphoreType.DMA((NSLOT,)),
            pltpu.SemaphoreType.DMA((NSLOT,)),
            pltpu.SemaphoreType.DMA((NSLOT,)),
            pltpu.SemaphoreType.DMA((NSLOT,)),
            pltpu.SemaphoreType.REGULAR((NSLOT,)),
            pltpu.SemaphoreType.REGULAR((NSLOT,)),
        ],
        compiler_params=pltpu.CompilerParams(
            collective_id=0, vmem_limit_bytes=52 * 1024 * 1024),
    )(qh, kv8, wo)


def kernel(x, Wq, Wk, Wv, Wo):
    my = lax.axis_index("i")
    x2 = x[0].astype(jnp.bfloat16)

    q = jnp.dot(x2, Wq.astype(jnp.bfloat16), preferred_element_type=jnp.float32)
    k = jnp.dot(x2, Wk.astype(jnp.bfloat16), preferred_element_type=jnp.float32)
    v = jnp.dot(x2, Wv.astype(jnp.bfloat16), preferred_element_type=jnp.float32)

    pos = (my * S_LOC + jnp.arange(S_LOC)).astype(jnp.float32)[:, None]
    inv = 1.0 / (10000.0 ** (jnp.arange(0, D, 2).astype(jnp.float32) / D))
    ang = pos * inv[None, :]
    cos = jnp.repeat(jnp.cos(ang), 2, axis=-1)
    sin = jnp.repeat(jnp.sin(ang), 2, axis=-1)

    def rot(t):
        t2 = t.reshape(S_LOC, H, D // 2, 2)
        t_r = jnp.stack([-t2[..., 1], t2[..., 0]], axis=-1).reshape(S_LOC, H, D)
        return t * cos[:, None, :] + t_r * sin[:, None, :]

    def to_i8(t):
        return jnp.clip(jnp.round(t / Q_SCALE), -127, 127).astype(jnp.int8)

    qh = (rot(q.reshape(S_LOC, H, D)) * (SCALE * Q_SCALE)
          ).transpose(1, 0, 2).astype(jnp.bfloat16)
    k8 = to_i8(rot(k.reshape(S_LOC, H, D)).transpose(1, 0, 2))
    v8 = to_i8(v.reshape(S_LOC, H, D).transpose(1, 0, 2))

    kv8 = jnp.concatenate([k8[:HH], v8[:HH], k8[HH:], v8[HH:]], axis=0)

    wo = (Wo * Q_SCALE).astype(jnp.bfloat16)
    out = _attention(qh, kv8, wo)
    return out.reshape(1, S_LOC, D_MODEL)
